# Initial kernel scaffold; baseline (speedup 1.0000x reference)
import jax
import jax.numpy as jnp
from jax.experimental import pallas as pl


def kernel(zs, pair_emb):
    raise NotImplementedError("write your pallas kernel here")


# speedup vs baseline: 359.8092x; 359.8092x over previous
import functools

import jax
import jax.numpy as jnp
from jax import lax
from jax.experimental import pallas as pl
from jax.experimental.pallas import tpu as pltpu
from jax.experimental.pallas import tpu_sc as plsc

NUM_T = 100
B = 32
L = 512
LANES = 16
NC, NS = 2, 16
ROW_CHUNK = 64
N_CHUNKS = L // ROW_CHUNK
JCHUNKS = L // LANES


def _pair_bias_body(zs_hbm, emb_hbm, out_hbm, zs_v, tab_v, buf0, buf1, sem0, sem1):
    b = lax.axis_index("s") * NC + lax.axis_index("c")

    pltpu.sync_copy(zs_hbm.at[b], zs_v.at[pl.ds(0, L)])
    pltpu.sync_copy(emb_hbm, tab_v)

    bufs = (buf0, buf1)
    sems = (sem0, sem1)

    def do_chunk(k, buf):
        def row_body(r, _):
            i = k * ROW_CHUNK + r
            base = zs_v[pl.ds(i, LANES)][0] * NUM_T
            for c in range(JCHUNKS):
                idx = zs_v[pl.ds(c * LANES, LANES)] + base
                vals = plsc.load_gather(tab_v, [idx])
                buf[pl.ds(r * L + c * LANES, LANES)] = vals
            return 0

        lax.fori_loop(0, ROW_CHUNK, row_body, 0)

    for k in range(N_CHUNKS):
        buf, sem = bufs[k % 2], sems[k % 2]
        if k >= 2:
            pltpu.make_async_copy(
                buf, out_hbm.at[b, pl.ds((k - 2) * ROW_CHUNK * L, ROW_CHUNK * L)], sem
            ).wait()
        do_chunk(k, buf)
        pltpu.async_copy(
            buf, out_hbm.at[b, pl.ds(k * ROW_CHUNK * L, ROW_CHUNK * L)], sem
        )
    for k in (N_CHUNKS - 2, N_CHUNKS - 1):
        buf, sem = bufs[k % 2], sems[k % 2]
        pltpu.make_async_copy(
            buf, out_hbm.at[b, pl.ds(k * ROW_CHUNK * L, ROW_CHUNK * L)], sem
        ).wait()


@functools.partial(jax.jit, static_argnames=())
def kernel(zs, pair_emb):
    zs32 = zs.astype(jnp.int32)
    emb = pair_emb.reshape(NUM_T * NUM_T)

    mesh = plsc.VectorSubcoreMesh(core_axis_name="c", subcore_axis_name="s")
    run = pl.kernel(
        _pair_bias_body,
        out_type=jax.ShapeDtypeStruct((B, L * L), jnp.float32),
        mesh=mesh,
        compiler_params=pltpu.CompilerParams(needs_layout_passes=False),
        scratch_types=[
            pltpu.VMEM((L + LANES,), jnp.int32),
            pltpu.VMEM((NUM_T * NUM_T,), jnp.float32),
            pltpu.VMEM((ROW_CHUNK * L,), jnp.float32),
            pltpu.VMEM((ROW_CHUNK * L,), jnp.float32),
            pltpu.SemaphoreType.DMA,
            pltpu.SemaphoreType.DMA,
        ],
    )
    out = run(zs32, emb)
    return out.reshape(B, L, L)


# speedup vs baseline: 1091.5197x; 3.0336x over previous
import functools

import jax
import jax.numpy as jnp
from jax import lax
from jax.experimental import pallas as pl
from jax.experimental.pallas import tpu as pltpu
from jax.experimental.pallas import tpu_sc as plsc

NUM_T = 100
B = 32
L = 512
LANES = 16
NC, NS = 2, 16
JCHUNKS = L // LANES
ROW_BYTES = L * 4


def _pair_bias_body(zs_hbm, emb_hbm, out_hbm, zs_v, tab_v, rtab_v, sem):
    b = lax.axis_index("s") * NC + lax.axis_index("c")

    pltpu.sync_copy(zs_hbm.at[b], zs_v)
    pltpu.sync_copy(emb_hbm, tab_v)

    zch = [zs_v[pl.ds(c * LANES, LANES)] for c in range(JCHUNKS)]

    def v_body(v, _):
        base = v * NUM_T
        for c in range(JCHUNKS):
            vals = plsc.load_gather(tab_v, [zch[c] + base])
            rtab_v[pl.ds(v * L + c * LANES, LANES)] = vals
        return 0

    lax.fori_loop(0, NUM_T, v_body, 0)

    def group_copies(g):
        rows = zs_v[pl.ds(g * LANES, LANES)]
        for r in range(LANES):
            v = rows[r]
            yield pltpu.make_async_copy(
                rtab_v.at[pl.ds(v * L, L)],
                out_hbm.at[b, pl.ds((g * LANES + r) * L, L)],
                sem,
            )

    def issue_group(g):
        for cp in group_copies(g):
            cp.start()

    def wait_group(g):
        for cp in group_copies(g):
            cp.wait()

    issue_group(0)

    def g_body(g, _):
        issue_group(g)
        wait_group(g - 1)
        return 0

    lax.fori_loop(1, L // LANES, g_body, 0)
    wait_group(L // LANES - 1)


@functools.partial(jax.jit, static_argnames=())
def kernel(zs, pair_emb):
    zs32 = zs.astype(jnp.int32)
    emb = pair_emb.reshape(NUM_T * NUM_T)

    mesh = plsc.VectorSubcoreMesh(core_axis_name="c", subcore_axis_name="s")
    run = pl.kernel(
        _pair_bias_body,
        out_type=jax.ShapeDtypeStruct((B, L * L), jnp.float32),
        mesh=mesh,
        compiler_params=pltpu.CompilerParams(needs_layout_passes=False),
        scratch_types=[
            pltpu.VMEM((L,), jnp.int32),
            pltpu.VMEM((NUM_T * NUM_T,), jnp.float32),
            pltpu.VMEM((NUM_T * L,), jnp.float32),
            pltpu.SemaphoreType.DMA,
        ],
    )
    out = run(zs32, emb)
    return out.reshape(B, L, L)


# speedup vs baseline: 1738.3020x; 1.5926x over previous
import functools

import jax
import jax.numpy as jnp
from jax import lax
from jax.experimental import pallas as pl
from jax.experimental.pallas import tpu as pltpu
from jax.experimental.pallas import tpu_sc as plsc

NUM_T = 100
B = 32
L = 512
LANES = 16
NC, NS = 2, 16
JCHUNKS = L // LANES
ROW_BYTES = L * 4


def _pair_bias_body(zs_hbm, emb_hbm, out_hbm, zs_v, tab_v, rtab_v, sem):
    b = lax.axis_index("s") * NC + lax.axis_index("c")

    pltpu.sync_copy(zs_hbm.at[b], zs_v)
    pltpu.sync_copy(emb_hbm, tab_v)

    zch = [zs_v[pl.ds(c * LANES, LANES)] for c in range(JCHUNKS)]

    def v_body(v, _):
        base = v * NUM_T
        for c in range(JCHUNKS):
            vals = plsc.load_gather(tab_v, [zch[c] + base])
            rtab_v[pl.ds(v * L + c * LANES, LANES)] = vals
        return 0

    lax.fori_loop(0, NUM_T, v_body, 0)

    def group_copies(g):
        rows = zs_v[pl.ds(g * LANES, LANES)]
        for r in range(LANES):
            v = rows[r]
            yield pltpu.make_async_copy(
                rtab_v.at[pl.ds(v * L, L)],
                out_hbm.at[b, g * LANES + r],
                sem,
            )

    def issue_group(g):
        for cp in group_copies(g):
            cp.start()

    def wait_group(g):
        for cp in group_copies(g):
            cp.wait()

    issue_group(0)

    def g_body(g, _):
        issue_group(g)
        wait_group(g - 1)
        return 0

    lax.fori_loop(1, L // LANES, g_body, 0)
    wait_group(L // LANES - 1)


@functools.partial(jax.jit, static_argnames=())
def kernel(zs, pair_emb):
    zs32 = zs.astype(jnp.int32)
    emb = pair_emb.reshape(NUM_T * NUM_T)

    mesh = plsc.VectorSubcoreMesh(core_axis_name="c", subcore_axis_name="s")
    run = pl.kernel(
        _pair_bias_body,
        out_type=jax.ShapeDtypeStruct((B, L, L), jnp.float32),
        mesh=mesh,
        compiler_params=pltpu.CompilerParams(needs_layout_passes=False),
        scratch_types=[
            pltpu.VMEM((L,), jnp.int32),
            pltpu.VMEM((NUM_T * NUM_T,), jnp.float32),
            pltpu.VMEM((NUM_T * L,), jnp.float32),
            pltpu.SemaphoreType.DMA,
        ],
    )
    return run(zs32, emb)


# speedup vs baseline: 1818.0962x; 1.0459x over previous
import functools

import jax
import jax.numpy as jnp
from jax import lax
from jax.experimental import pallas as pl
from jax.experimental.pallas import tpu as pltpu
from jax.experimental.pallas import tpu_sc as plsc

NUM_T = 100
B = 32
L = 512
LANES = 16
NC, NS = 2, 16
JCHUNKS = L // LANES
ROW_BYTES = L * 4


def _pair_bias_body(zs_hbm, emb_hbm, out_hbm, zs_v, tab_v, rtab_v, sem):
    b = lax.axis_index("s") * NC + lax.axis_index("c")

    pltpu.sync_copy(zs_hbm.at[b], zs_v)
    pltpu.sync_copy(emb_hbm, tab_v)

    zch = [zs_v[pl.ds(c * LANES, LANES)] for c in range(JCHUNKS)]

    @plsc.parallel_loop(0, NUM_T, unroll=2)
    def v_body(v):
        base = v * NUM_T
        for c in range(JCHUNKS):
            vals = plsc.load_gather(tab_v, [zch[c] + base])
            rtab_v[pl.ds(v * L + c * LANES, LANES)] = vals

    def group_copies(g):
        rows = zs_v[pl.ds(g * LANES, LANES)]
        for r in range(LANES):
            v = rows[r]
            yield pltpu.make_async_copy(
                rtab_v.at[pl.ds(v * L, L)],
                out_hbm.at[b, g * LANES + r],
                sem,
            )

    def issue_group(g):
        for cp in group_copies(g):
            cp.start()

    def wait_group(g):
        for cp in group_copies(g):
            cp.wait()

    issue_group(0)
    issue_group(1)

    def g_body(g, _):
        issue_group(g)
        wait_group(g - 2)
        return 0

    lax.fori_loop(2, L // LANES, g_body, 0)
    wait_group(L // LANES - 2)
    wait_group(L // LANES - 1)


@functools.partial(jax.jit, static_argnames=())
def kernel(zs, pair_emb):
    zs32 = zs.astype(jnp.int32)
    emb = pair_emb.reshape(NUM_T * NUM_T)

    mesh = plsc.VectorSubcoreMesh(core_axis_name="c", subcore_axis_name="s")
    run = pl.kernel(
        _pair_bias_body,
        out_type=jax.ShapeDtypeStruct((B, L, L), jnp.float32),
        mesh=mesh,
        compiler_params=pltpu.CompilerParams(needs_layout_passes=False),
        scratch_types=[
            pltpu.VMEM((L,), jnp.int32),
            pltpu.VMEM((NUM_T * NUM_T,), jnp.float32),
            pltpu.VMEM((NUM_T * L,), jnp.float32),
            pltpu.SemaphoreType.DMA,
        ],
    )
    return run(zs32, emb)


# speedup vs baseline: 2210.1077x; 1.2156x over previous
import functools

import jax
import jax.numpy as jnp
from jax import lax
from jax.experimental import pallas as pl
from jax.experimental.pallas import tpu as pltpu
from jax.experimental.pallas import tpu_sc as plsc

NUM_T = 100
B = 32
L = 512
LANES = 16
NC, NS = 2, 16
JCHUNKS = L // LANES
ROW_BYTES = L * 4


def _pair_bias_body(zs_hbm, emb_hbm, out_hbm, zs_v, tab_v, rtab_v, sem):
    b = lax.axis_index("s") * NC + lax.axis_index("c")

    pltpu.sync_copy(zs_hbm.at[b], zs_v)
    pltpu.sync_copy(emb_hbm, tab_v)

    zch = [zs_v[pl.ds(c * LANES, LANES)] for c in range(JCHUNKS)]

    @plsc.parallel_loop(0, NUM_T, unroll=4)
    def v_body(v):
        base = v * NUM_T
        for c in range(JCHUNKS):
            vals = plsc.load_gather(tab_v, [zch[c] + base])
            rtab_v[pl.ds(v * L + c * LANES, LANES)] = vals

    def group_copies(g):
        rows = zs_v[pl.ds(g * LANES, LANES)]
        for r in range(LANES):
            v = rows[r]
            yield pltpu.make_async_copy(
                rtab_v.at[pl.ds(v * L, L)],
                out_hbm.at[b, g * LANES + r],
                sem,
            )

    def issue_group(g):
        for cp in group_copies(g):
            cp.start()

    def wait_group(g):
        for cp in group_copies(g):
            cp.wait()

    issue_group(0)
    issue_group(1)

    def g_body(g, _):
        issue_group(g)
        wait_group(g - 2)
        return 0

    lax.fori_loop(2, L // LANES, g_body, 0)
    wait_group(L // LANES - 2)
    wait_group(L // LANES - 1)


@functools.partial(jax.jit, static_argnames=())
def kernel(zs, pair_emb):
    zs32 = zs.astype(jnp.int32)
    emb = pair_emb.reshape(NUM_T * NUM_T)

    mesh = plsc.VectorSubcoreMesh(core_axis_name="c", subcore_axis_name="s")
    run = pl.kernel(
        _pair_bias_body,
        out_type=jax.ShapeDtypeStruct((B, L, L), jnp.float32),
        mesh=mesh,
        compiler_params=pltpu.CompilerParams(needs_layout_passes=False),
        scratch_types=[
            pltpu.VMEM((L,), jnp.int32),
            pltpu.VMEM((NUM_T * NUM_T,), jnp.float32),
            pltpu.VMEM((NUM_T * L,), jnp.float32),
            pltpu.SemaphoreType.DMA,
        ],
    )
    return run(zs32, emb)
